# baseline (speedup 1.0000x reference)
import functools

import jax
import jax.numpy as jnp
from jax import lax
from jax.experimental import pallas as pl
from jax.experimental.pallas import tpu as pltpu
from jax.experimental.pallas import tpu_sc as plsc

_G = 17911
_B, _N, _D = 4, 8192, 768
_EMB = 128
_TOP = 5000
_TOPP = 5120
_RTOT = _B * _TOPP

def _mono(v):
    b = lax.bitcast_convert_type(v, jnp.int32)
    return jnp.where(b >= 0, b, jnp.full_like(b, 2147483647) - b)


_BI = 512
_BJ = 512


def _rank_body(xi_ref, xrow_ref, o_ref):
    g = pl.program_id(1)
    mi_row = _mono(xi_ref[0, 0, :])
    mi = mi_row[:, None]

    def lower(jc, acc):
        mj = _mono(xrow_ref[0, 0, pl.ds(jc * _BJ, _BJ)])[None, :]
        return acc + jnp.sum((mj >= mi).astype(jnp.float32), axis=1)

    def upper(jc, acc):
        mj = _mono(xrow_ref[0, 0, pl.ds(jc * _BJ, _BJ)])[None, :]
        return acc + jnp.sum((mj > mi).astype(jnp.float32), axis=1)

    acc = lax.fori_loop(0, g, lower, jnp.zeros((_BI,), jnp.float32))
    acc = lax.fori_loop(g + 1, _N // _BJ, upper, acc)
    mjd = mi_row[None, :]
    ii = lax.broadcasted_iota(jnp.int32, (_BI, _BI), 0)
    jj = lax.broadcasted_iota(jnp.int32, (_BI, _BI), 1)
    cmp = (mjd > mi) | ((mjd == mi) & (jj < ii))
    acc = acc + jnp.sum(cmp.astype(jnp.float32), axis=1)
    o_ref[0, 0, :] = acc.astype(jnp.int32)


def _rank_call(x_raw):
    x3 = x_raw.reshape(_B, 1, _N)
    out = pl.pallas_call(
        _rank_body,
        grid=(_B, _N // _BI),
        in_specs=[
            pl.BlockSpec((1, 1, _BI), lambda b, g: (b, 0, g)),
            pl.BlockSpec((1, 1, _N), lambda b, g: (b, 0, 0)),
        ],
        out_specs=pl.BlockSpec((1, 1, _BI), lambda b, g: (b, 0, g)),
        out_shape=jax.ShapeDtypeStruct((_B, 1, _N), jnp.int32),
    )(x3, x3)
    return out.reshape(_B, _N)


_GB = 1024


def _fc1_body(grn_ref, ppi_ref, w_ref, b_ref, o_ref):
    w = w_ref[...]
    o_ref[...] = (
        jnp.dot(grn_ref[...], w[:_EMB, :], preferred_element_type=jnp.float32)
        + jnp.dot(ppi_ref[...], w[_EMB:, :], preferred_element_type=jnp.float32)
        + b_ref[...]
    )


def _fc1_call(grn, ppi, w, b2d):
    ng = (_G + _GB - 1) // _GB
    return pl.pallas_call(
        _fc1_body,
        grid=(ng,),
        in_specs=[
            pl.BlockSpec((_GB, _EMB), lambda i: (i, 0)),
            pl.BlockSpec((_GB, _EMB), lambda i: (i, 0)),
            pl.BlockSpec((2 * _EMB, _EMB), lambda i: (0, 0)),
            pl.BlockSpec((1, _EMB), lambda i: (0, 0)),
        ],
        out_specs=pl.BlockSpec((_GB, _EMB), lambda i: (i, 0)),
        out_shape=jax.ShapeDtypeStruct((_G, _EMB), jnp.float32),
    )(grn, ppi, w, b2d)


_CH = 256
_PC = 128
_PCX = 64


def _sc_select_gather(ranks, xraw, xind, xflat, fc1):
    mesh = plsc.VectorSubcoreMesh(core_axis_name="c", subcore_axis_name="s")

    @functools.partial(
        pl.kernel,
        mesh=mesh,
        compiler_params=pltpu.CompilerParams(needs_layout_passes=False),
        out_type=[
            jax.ShapeDtypeStruct((_RTOT, _D), jnp.float32),
            jax.ShapeDtypeStruct((_RTOT, _EMB), jnp.float32),
            jax.ShapeDtypeStruct((_RTOT,), jnp.float32),
            jax.ShapeDtypeStruct((_RTOT,), jnp.int32),
            jax.ShapeDtypeStruct((_RTOT,), jnp.int32),
        ],
        scratch_types=[
            pltpu.VMEM((_CH,), jnp.int32),
            pltpu.VMEM((_CH,), jnp.float32),
            pltpu.VMEM((_CH,), jnp.int32),
            pltpu.VMEM((_TOPP,), jnp.int32),
            pltpu.VMEM((_TOPP,), jnp.float32),
            pltpu.VMEM((_TOPP,), jnp.int32),
            pltpu.VMEM((_PCX,), jnp.int32),
            pltpu.VMEM((_PC,), jnp.int32),
            pltpu.VMEM((_PCX, _D), jnp.float32),
            pltpu.VMEM((_PC, _EMB), jnp.float32),
            pltpu.SemaphoreType.DMA,
        ],
    )
    def k(ranks_h, xraw_h, xind_h, xflat_h, fc1_h,
          xg_h, sel_h, rg_h, tixs_h, tgis_h,
          rank_c, val_c, gid_c, tix, trg, tgi, idx_a, gidx, xrow, selb, sem):
        core = lax.axis_index("c")
        s = lax.axis_index("s")

        @pl.when(s < 2)
        def phase1():
            b = core * 2 + s
            base = b * _N

            def initf(i, carry):
                z = jnp.zeros((16,), jnp.int32)
                tix[pl.ds(i * 16, 16)] = z
                tgi[pl.ds(i * 16, 16)] = z
                return carry

            lax.fori_loop(0, _TOPP // 16, initf, 0)

            def outer(cc, carry):
                off = base + cc * _CH
                pltpu.sync_copy(ranks_h.at[pl.ds(off, _CH)], rank_c)
                pltpu.sync_copy(xraw_h.at[pl.ds(off, _CH)], val_c)
                pltpu.sync_copy(xind_h.at[pl.ds(off, _CH)], gid_c)

                def inner(kk, carry2):
                    rv = rank_c[pl.ds(kk * 16, 16)]
                    vv = val_c[pl.ds(kk * 16, 16)]
                    gv = gid_c[pl.ds(kk * 16, 16)]
                    ig = (off + kk * 16
                          + lax.broadcasted_iota(jnp.int32, (16,), 0))
                    m = rv < _TOP
                    plsc.store_scatter(tix, [rv], ig, mask=m)
                    plsc.store_scatter(trg, [rv], vv, mask=m)
                    plsc.store_scatter(tgi, [rv], gv, mask=m)
                    return carry2

                lax.fori_loop(0, _CH // 16, inner, 0)
                return carry

            lax.fori_loop(0, _N // _CH, outer, 0)

            stage = core * (2 * _TOPP) + s * _TOPP
            pltpu.sync_copy(tix, tixs_h.at[pl.ds(stage, _TOPP)])
            pltpu.sync_copy(tgi, tgis_h.at[pl.ds(stage, _TOPP)])
            pltpu.sync_copy(trg, rg_h.at[pl.ds(b * _TOPP, _TOPP)])

        plsc.subcore_barrier()

        nch = (2 * _TOPP) // _PC // 16
        for q in range(nch):
            c = s * nch + q
            off = core * (2 * _TOPP) + c * _PC
            pltpu.sync_copy(tgis_h.at[pl.ds(off, _PC)], gidx)
            pltpu.async_copy(fc1_h.at[gidx], selb, sem).wait()
            pltpu.sync_copy(selb, sel_h.at[pl.ds(off, _PC)])
            for h in range(_PC // _PCX):
                pltpu.sync_copy(tixs_h.at[pl.ds(off + h * _PCX, _PCX)], idx_a)
                pltpu.async_copy(xflat_h.at[idx_a], xrow, sem).wait()
                pltpu.sync_copy(xrow, xg_h.at[pl.ds(off + h * _PCX, _PCX)])

    return k(ranks, xraw, xind, xflat, fc1)


_RB = 512


def _mlp_body(xg_ref, sel_ref, rg_ref, wx_ref, ws_ref, wr_ref, cb_ref,
              t1w_ref, t1b_ref, t2w_ref, t2b_ref, lng_ref, lnb_ref,
              pw_ref, pb_ref, o_ref):
    r = rg_ref[...]
    h1 = jnp.maximum(r * t1w_ref[...] + t1b_ref[...], 0.0)
    remb = (jnp.dot(h1, t2w_ref[...], preferred_element_type=jnp.float32)
            + t2b_ref[...])
    h2 = (jnp.dot(xg_ref[...], wx_ref[...], preferred_element_type=jnp.float32)
          + jnp.dot(sel_ref[...], ws_ref[...], preferred_element_type=jnp.float32)
          + jnp.dot(remb, wr_ref[...], preferred_element_type=jnp.float32)
          + cb_ref[...])
    mu = jnp.mean(h2, axis=1, keepdims=True)
    d0 = h2 - mu
    var = jnp.mean(d0 * d0, axis=1, keepdims=True)
    hn = d0 * lax.rsqrt(var + 1e-5) * lng_ref[...] + lnb_ref[...]
    hg = hn * (1.0 / (1.0 + jnp.exp(-1.702 * hn)))
    o_ref[...] = (jnp.dot(hg, pw_ref[...], preferred_element_type=jnp.float32)
                  + pb_ref[...])


def _mlp_call(xg, sel, rg2d, wx, ws, wr, cb, t1w, t1b, t2w, t2b,
              lng, lnb, pw, pb):
    full = lambda shape: pl.BlockSpec(shape, lambda i: tuple(0 for _ in shape))
    return pl.pallas_call(
        _mlp_body,
        grid=(_RTOT // _RB,),
        in_specs=[
            pl.BlockSpec((_RB, _D), lambda i: (i, 0)),
            pl.BlockSpec((_RB, _EMB), lambda i: (i, 0)),
            pl.BlockSpec((_RB, 1), lambda i: (i, 0)),
            full((_D, _D)),
            full((_EMB, _D)),
            full((_EMB, _D)),
            full((1, _D)),
            full((1, 50)),
            full((1, 50)),
            full((50, _EMB)),
            full((1, _EMB)),
            full((1, _D)),
            full((1, _D)),
            full((_D, _D)),
            full((1, _D)),
        ],
        out_specs=pl.BlockSpec((_RB, _D), lambda i: (i, 0)),
        out_shape=jax.ShapeDtypeStruct((_RTOT, _D), jnp.float32),
    )(xg, sel, rg2d, wx, ws, wr, cb, t1w, t1b, t2w, t2b, lng, lnb, pw, pb)


def kernel(x, x_raw, x_indices, grn_emb, ppi_emb, fc1_w, fc1_b, t1_w, t1_b,
           t2_w, t2_b, cat_fc_w, cat_fc_b, ln_g, ln_b, proj_w, proj_b):
    ranks = _rank_call(x_raw)
    fc1_out = _fc1_call(grn_emb, ppi_emb, fc1_w, fc1_b.reshape(1, _EMB))
    xg, sel, rg, _, _ = _sc_select_gather(
        ranks.reshape(-1),
        x_raw.reshape(-1),
        x_indices.reshape(-1),
        x.reshape(_B * _N, _D),
        fc1_out,
    )
    y = _mlp_call(
        xg, sel, rg.reshape(_RTOT, 1),
        cat_fc_w[:_D], cat_fc_w[_D:_D + _EMB], cat_fc_w[_D + _EMB:],
        cat_fc_b.reshape(1, _D), t1_w, t1_b.reshape(1, 50), t2_w,
        t2_b.reshape(1, _EMB), ln_g.reshape(1, _D), ln_b.reshape(1, _D),
        proj_w, proj_b.reshape(1, _D),
    )
    out = jnp.stack(
        [lax.slice(y, (b * _TOPP, 0), (b * _TOPP + _TOP, _D))
         for b in range(_B)])
    return out


# speedup vs baseline: 1.0848x; 1.0848x over previous
import functools

import jax
import jax.numpy as jnp
from jax import lax
from jax.experimental import pallas as pl
from jax.experimental.pallas import tpu as pltpu
from jax.experimental.pallas import tpu_sc as plsc

_G = 17911
_B, _N, _D = 4, 8192, 768
_EMB = 128
_TOP = 5000
_TOPP = 5120
_RTOT = _B * _TOPP

def _mono(v):
    b = lax.bitcast_convert_type(v, jnp.int32)
    return jnp.where(b >= 0, b, jnp.full_like(b, 2147483647) - b)


_BI = 512
_BJ = 512


def _rank_body(xi_ref, xrow_ref, o_ref):
    g = pl.program_id(1)
    mi_row = _mono(xi_ref[0, 0, :])
    mi = mi_row[:, None]

    def lower(jc, acc):
        mj = _mono(xrow_ref[0, 0, pl.ds(jc * _BJ, _BJ)])[None, :]
        return acc + jnp.sum((mj >= mi).astype(jnp.float32), axis=1)

    def upper(jc, acc):
        mj = _mono(xrow_ref[0, 0, pl.ds(jc * _BJ, _BJ)])[None, :]
        return acc + jnp.sum((mj > mi).astype(jnp.float32), axis=1)

    acc = lax.fori_loop(0, g, lower, jnp.zeros((_BI,), jnp.float32))
    acc = lax.fori_loop(g + 1, _N // _BJ, upper, acc)
    mjd = mi_row[None, :]
    ii = lax.broadcasted_iota(jnp.int32, (_BI, _BI), 0)
    jj = lax.broadcasted_iota(jnp.int32, (_BI, _BI), 1)
    cmp = (mjd > mi) | ((mjd == mi) & (jj < ii))
    acc = acc + jnp.sum(cmp.astype(jnp.float32), axis=1)
    o_ref[0, 0, :] = acc.astype(jnp.int32)


def _rank_call(x_raw):
    x3 = x_raw.reshape(_B, 1, _N)
    out = pl.pallas_call(
        _rank_body,
        grid=(_B, _N // _BI),
        in_specs=[
            pl.BlockSpec((1, 1, _BI), lambda b, g: (b, 0, g)),
            pl.BlockSpec((1, 1, _N), lambda b, g: (b, 0, 0)),
        ],
        out_specs=pl.BlockSpec((1, 1, _BI), lambda b, g: (b, 0, g)),
        out_shape=jax.ShapeDtypeStruct((_B, 1, _N), jnp.int32),
    )(x3, x3)
    return out.reshape(_B, _N)


_GB = 1024


def _fc1_body(grn_ref, ppi_ref, w_ref, b_ref, o_ref):
    w = w_ref[...]
    o_ref[...] = (
        jnp.dot(grn_ref[...], w[:_EMB, :], preferred_element_type=jnp.float32)
        + jnp.dot(ppi_ref[...], w[_EMB:, :], preferred_element_type=jnp.float32)
        + b_ref[...]
    )


def _fc1_call(grn, ppi, w, b2d):
    ng = (_G + _GB - 1) // _GB
    return pl.pallas_call(
        _fc1_body,
        grid=(ng,),
        in_specs=[
            pl.BlockSpec((_GB, _EMB), lambda i: (i, 0)),
            pl.BlockSpec((_GB, _EMB), lambda i: (i, 0)),
            pl.BlockSpec((2 * _EMB, _EMB), lambda i: (0, 0)),
            pl.BlockSpec((1, _EMB), lambda i: (0, 0)),
        ],
        out_specs=pl.BlockSpec((_GB, _EMB), lambda i: (i, 0)),
        out_shape=jax.ShapeDtypeStruct((_G, _EMB), jnp.float32),
    )(grn, ppi, w, b2d)


_CH = 2048
_PC = 128
_PCX = 40


def _sc_select_gather(ranks, xraw, xind, xflat, fc1):
    mesh = plsc.VectorSubcoreMesh(core_axis_name="c", subcore_axis_name="s")
    rows_per_tile = (2 * _TOPP) // 16

    @functools.partial(
        pl.kernel,
        mesh=mesh,
        compiler_params=pltpu.CompilerParams(needs_layout_passes=False),
        out_type=[
            jax.ShapeDtypeStruct((_RTOT, _D), jnp.float32),
            jax.ShapeDtypeStruct((_RTOT, _EMB), jnp.float32),
            jax.ShapeDtypeStruct((_RTOT,), jnp.float32),
            jax.ShapeDtypeStruct((_RTOT,), jnp.int32),
            jax.ShapeDtypeStruct((_RTOT,), jnp.int32),
        ],
        scratch_types=[
            pltpu.VMEM((_CH,), jnp.int32),
            pltpu.VMEM((_CH,), jnp.float32),
            pltpu.VMEM((_CH,), jnp.int32),
            pltpu.VMEM((_TOPP,), jnp.int32),
            pltpu.VMEM((_TOPP,), jnp.float32),
            pltpu.VMEM((_TOPP,), jnp.int32),
            pltpu.VMEM((640,), jnp.int32),
            pltpu.VMEM((640,), jnp.int32),
            pltpu.VMEM((_PCX, _D), jnp.float32),
            pltpu.VMEM((_PCX, _D), jnp.float32),
            pltpu.VMEM((_PC, _EMB), jnp.float32),
            pltpu.VMEM((_PC, _EMB), jnp.float32),
            pltpu.SemaphoreType.DMA,
            pltpu.SemaphoreType.DMA,
            pltpu.SemaphoreType.DMA,
        ],
    )
    def k(ranks_h, xraw_h, xind_h, xflat_h, fc1_h,
          xg_h, sel_h, rg_h, tixs_h, tgis_h,
          rank_c, val_c, gid_c, tix, trg, tgi, idx_all, gidx_all,
          xrow_a, xrow_b, sel_a, sel_b, sem_a, sem_b, sem_c):
        core = lax.axis_index("c")
        s = lax.axis_index("s")

        @pl.when(s < 2)
        def phase1():
            b = core * 2 + s
            base = b * _N

            def initf(i, carry):
                z = jnp.zeros((16,), jnp.int32)
                tix[pl.ds(i * 16, 16)] = z
                tgi[pl.ds(i * 16, 16)] = z
                return carry

            lax.fori_loop(0, _TOPP // 16, initf, 0)

            def outer(cc, carry):
                off = base + cc * _CH
                pltpu.sync_copy(ranks_h.at[pl.ds(off, _CH)], rank_c)
                pltpu.sync_copy(xraw_h.at[pl.ds(off, _CH)], val_c)
                pltpu.sync_copy(xind_h.at[pl.ds(off, _CH)], gid_c)

                def inner(kk, carry2):
                    rv = rank_c[pl.ds(kk * 16, 16)]
                    vv = val_c[pl.ds(kk * 16, 16)]
                    gv = gid_c[pl.ds(kk * 16, 16)]
                    ig = (off + kk * 16
                          + lax.broadcasted_iota(jnp.int32, (16,), 0))
                    m = rv < _TOP
                    plsc.store_scatter(tix, [rv], ig, mask=m)
                    plsc.store_scatter(trg, [rv], vv, mask=m)
                    plsc.store_scatter(tgi, [rv], gv, mask=m)
                    return carry2

                lax.fori_loop(0, _CH // 16, inner, 0)
                return carry

            lax.fori_loop(0, _N // _CH, outer, 0)

            stage = core * (2 * _TOPP) + s * _TOPP
            pltpu.sync_copy(tix, tixs_h.at[pl.ds(stage, _TOPP)])
            pltpu.sync_copy(tgi, tgis_h.at[pl.ds(stage, _TOPP)])
            pltpu.sync_copy(trg, rg_h.at[pl.ds(b * _TOPP, _TOPP)])

        plsc.subcore_barrier()

        tbase = core * (2 * _TOPP) + s * rows_per_tile
        pltpu.sync_copy(tixs_h.at[pl.ds(tbase, rows_per_tile)], idx_all)
        pltpu.sync_copy(tgis_h.at[pl.ds(tbase, rows_per_tile)], gidx_all)

        nx = rows_per_tile // _PCX
        xbufs = (xrow_a, xrow_b)
        xsems = (sem_a, sem_b)
        cps = [None, None]
        cps[0] = pltpu.async_copy(
            xflat_h.at[idx_all.at[pl.ds(0, _PCX)]], xrow_a, sem_a)
        for i in range(nx):
            bsl = i % 2
            if i + 1 < nx:
                cps[(i + 1) % 2] = pltpu.async_copy(
                    xflat_h.at[idx_all.at[pl.ds((i + 1) * _PCX, _PCX)]],
                    xbufs[(i + 1) % 2], xsems[(i + 1) % 2])
            cps[bsl].wait()
            pltpu.sync_copy(xbufs[bsl],
                            xg_h.at[pl.ds(tbase + i * _PCX, _PCX)])

        ns = rows_per_tile // _PC
        sbufs = (sel_a, sel_b)
        ssems = (sem_c, sem_a)
        scps = [None, None]
        scps[0] = pltpu.async_copy(
            fc1_h.at[gidx_all.at[pl.ds(0, _PC)]], sel_a, sem_c)
        for i in range(ns):
            bsl = i % 2
            if i + 1 < ns:
                scps[(i + 1) % 2] = pltpu.async_copy(
                    fc1_h.at[gidx_all.at[pl.ds((i + 1) * _PC, _PC)]],
                    sbufs[(i + 1) % 2], ssems[(i + 1) % 2])
            scps[bsl].wait()
            pltpu.sync_copy(sbufs[bsl],
                            sel_h.at[pl.ds(tbase + i * _PC, _PC)])

    return k(ranks, xraw, xind, xflat, fc1)


_RB = 512


def _mlp_body(xg_ref, sel_ref, rg_ref, wx_ref, ws_ref, wr_ref, cb_ref,
              t1w_ref, t1b_ref, t2w_ref, t2b_ref, lng_ref, lnb_ref,
              pw_ref, pb_ref, o_ref):
    r = rg_ref[...]
    h1 = jnp.maximum(r * t1w_ref[...] + t1b_ref[...], 0.0)
    remb = (jnp.dot(h1, t2w_ref[...], preferred_element_type=jnp.float32)
            + t2b_ref[...])
    h2 = (jnp.dot(xg_ref[...], wx_ref[...], preferred_element_type=jnp.float32)
          + jnp.dot(sel_ref[...], ws_ref[...], preferred_element_type=jnp.float32)
          + jnp.dot(remb, wr_ref[...], preferred_element_type=jnp.float32)
          + cb_ref[...])
    mu = jnp.mean(h2, axis=1, keepdims=True)
    d0 = h2 - mu
    var = jnp.mean(d0 * d0, axis=1, keepdims=True)
    hn = d0 * lax.rsqrt(var + 1e-5) * lng_ref[...] + lnb_ref[...]
    hg = hn * (1.0 / (1.0 + jnp.exp(-1.702 * hn)))
    o_ref[...] = (jnp.dot(hg, pw_ref[...], preferred_element_type=jnp.float32)
                  + pb_ref[...])


def _mlp_call(xg, sel, rg2d, wx, ws, wr, cb, t1w, t1b, t2w, t2b,
              lng, lnb, pw, pb):
    full = lambda shape: pl.BlockSpec(shape, lambda i: tuple(0 for _ in shape))
    return pl.pallas_call(
        _mlp_body,
        grid=(_RTOT // _RB,),
        in_specs=[
            pl.BlockSpec((_RB, _D), lambda i: (i, 0)),
            pl.BlockSpec((_RB, _EMB), lambda i: (i, 0)),
            pl.BlockSpec((_RB, 1), lambda i: (i, 0)),
            full((_D, _D)),
            full((_EMB, _D)),
            full((_EMB, _D)),
            full((1, _D)),
            full((1, 50)),
            full((1, 50)),
            full((50, _EMB)),
            full((1, _EMB)),
            full((1, _D)),
            full((1, _D)),
            full((_D, _D)),
            full((1, _D)),
        ],
        out_specs=pl.BlockSpec((_RB, _D), lambda i: (i, 0)),
        out_shape=jax.ShapeDtypeStruct((_RTOT, _D), jnp.float32),
    )(xg, sel, rg2d, wx, ws, wr, cb, t1w, t1b, t2w, t2b, lng, lnb, pw, pb)


def kernel(x, x_raw, x_indices, grn_emb, ppi_emb, fc1_w, fc1_b, t1_w, t1_b,
           t2_w, t2_b, cat_fc_w, cat_fc_b, ln_g, ln_b, proj_w, proj_b):
    ranks = _rank_call(x_raw)
    fc1_out = _fc1_call(grn_emb, ppi_emb, fc1_w, fc1_b.reshape(1, _EMB))
    xg, sel, rg, _, _ = _sc_select_gather(
        ranks.reshape(-1),
        x_raw.reshape(-1),
        x_indices.reshape(-1),
        x.reshape(_B * _N, _D),
        fc1_out,
    )
    y = _mlp_call(
        xg, sel, rg.reshape(_RTOT, 1),
        cat_fc_w[:_D], cat_fc_w[_D:_D + _EMB], cat_fc_w[_D + _EMB:],
        cat_fc_b.reshape(1, _D), t1_w, t1_b.reshape(1, 50), t2_w,
        t2_b.reshape(1, _EMB), ln_g.reshape(1, _D), ln_b.reshape(1, _D),
        proj_w, proj_b.reshape(1, _D),
    )
    out = jnp.stack(
        [lax.slice(y, (b * _TOPP, 0), (b * _TOPP + _TOP, _D))
         for b in range(_B)])
    return out


# speedup vs baseline: 1.3675x; 1.2606x over previous
import functools

import jax
import jax.numpy as jnp
from jax import lax
from jax.experimental import pallas as pl
from jax.experimental.pallas import tpu as pltpu
from jax.experimental.pallas import tpu_sc as plsc

_G = 17911
_B, _N, _D = 4, 8192, 768
_EMB = 128
_TOP = 5000
_RTOT = _B * _TOP

def _mono(v):
    b = lax.bitcast_convert_type(v, jnp.int32)
    return jnp.where(b >= 0, b, jnp.full_like(b, 2147483647) - b)


_BI = 512
_BJ = 512


def _rank_body(xi_ref, xrow_ref, o_ref):
    g = pl.program_id(1)
    mi_row = _mono(xi_ref[0, 0, :])
    mi = mi_row[:, None]

    def lower(jc, acc):
        mj = _mono(xrow_ref[0, 0, pl.ds(jc * _BJ, _BJ)])[None, :]
        return acc + jnp.sum((mj >= mi).astype(jnp.float32), axis=1)

    def upper(jc, acc):
        mj = _mono(xrow_ref[0, 0, pl.ds(jc * _BJ, _BJ)])[None, :]
        return acc + jnp.sum((mj > mi).astype(jnp.float32), axis=1)

    acc = lax.fori_loop(0, g, lower, jnp.zeros((_BI,), jnp.float32))
    acc = lax.fori_loop(g + 1, _N // _BJ, upper, acc)
    mjd = mi_row[None, :]
    ii = lax.broadcasted_iota(jnp.int32, (_BI, _BI), 0)
    jj = lax.broadcasted_iota(jnp.int32, (_BI, _BI), 1)
    cmp = (mjd > mi) | ((mjd == mi) & (jj < ii))
    acc = acc + jnp.sum(cmp.astype(jnp.float32), axis=1)
    o_ref[0, 0, :] = acc.astype(jnp.int32)


def _rank_call(x_raw):
    x3 = x_raw.reshape(_B, 1, _N)
    out = pl.pallas_call(
        _rank_body,
        grid=(_B, _N // _BI),
        in_specs=[
            pl.BlockSpec((1, 1, _BI), lambda b, g: (b, 0, g)),
            pl.BlockSpec((1, 1, _N), lambda b, g: (b, 0, 0)),
        ],
        out_specs=pl.BlockSpec((1, 1, _BI), lambda b, g: (b, 0, g)),
        out_shape=jax.ShapeDtypeStruct((_B, 1, _N), jnp.int32),
    )(x3, x3)
    return out.reshape(_B, _N)


_GB = 1024


def _fc1_body(grn_ref, ppi_ref, w_ref, b_ref, o_ref):
    w = w_ref[...]
    o_ref[...] = (
        jnp.dot(grn_ref[...], w[:_EMB, :], preferred_element_type=jnp.float32)
        + jnp.dot(ppi_ref[...], w[_EMB:, :], preferred_element_type=jnp.float32)
        + b_ref[...]
    )


def _fc1_call(grn, ppi, w, b2d):
    ng = (_G + _GB - 1) // _GB
    return pl.pallas_call(
        _fc1_body,
        grid=(ng,),
        in_specs=[
            pl.BlockSpec((_GB, _EMB), lambda i: (i, 0)),
            pl.BlockSpec((_GB, _EMB), lambda i: (i, 0)),
            pl.BlockSpec((2 * _EMB, _EMB), lambda i: (0, 0)),
            pl.BlockSpec((1, _EMB), lambda i: (0, 0)),
        ],
        out_specs=pl.BlockSpec((_GB, _EMB), lambda i: (i, 0)),
        out_shape=jax.ShapeDtypeStruct((_G, _EMB), jnp.float32),
    )(grn, ppi, w, b2d)


_CH = 2048
_PC = 200
_PCX = 40


def _sc_select_gather(ranks, xraw, xind, xflat, fc1):
    mesh = plsc.VectorSubcoreMesh(core_axis_name="c", subcore_axis_name="s")
    core_rows = 2 * _TOP
    nchunks = core_rows // _PC

    @functools.partial(
        pl.kernel,
        mesh=mesh,
        compiler_params=pltpu.CompilerParams(needs_layout_passes=False),
        out_type=[
            jax.ShapeDtypeStruct((_RTOT, _D), jnp.float32),
            jax.ShapeDtypeStruct((_RTOT, _EMB), jnp.float32),
            jax.ShapeDtypeStruct((_RTOT,), jnp.float32),
            jax.ShapeDtypeStruct((_RTOT,), jnp.int32),
            jax.ShapeDtypeStruct((_RTOT,), jnp.int32),
        ],
        scratch_types=[
            pltpu.VMEM((_CH,), jnp.int32),
            pltpu.VMEM((_CH,), jnp.float32),
            pltpu.VMEM((_CH,), jnp.int32),
            pltpu.VMEM((_TOP,), jnp.int32),
            pltpu.VMEM((_TOP,), jnp.float32),
            pltpu.VMEM((_TOP,), jnp.int32),
            pltpu.VMEM((_PC,), jnp.int32),
            pltpu.VMEM((_PC,), jnp.int32),
            pltpu.VMEM((_PCX, _D), jnp.float32),
            pltpu.VMEM((_PCX, _D), jnp.float32),
            pltpu.VMEM((_PC, _EMB), jnp.float32),
            pltpu.SemaphoreType.DMA,
            pltpu.SemaphoreType.DMA,
            pltpu.SemaphoreType.DMA,
        ],
    )
    def k(ranks_h, xraw_h, xind_h, xflat_h, fc1_h,
          xg_h, sel_h, rg_h, tixs_h, tgis_h,
          rank_c, val_c, gid_c, tix, trg, tgi, idx_c, gidx_c,
          xrow_a, xrow_b, selb, sem_a, sem_b, sem_c):
        core = lax.axis_index("c")
        s = lax.axis_index("s")

        @pl.when(s < 2)
        def phase1():
            b = core * 2 + s
            base = b * _N

            def outer(cc, carry):
                off = base + cc * _CH
                pltpu.sync_copy(ranks_h.at[pl.ds(off, _CH)], rank_c)
                pltpu.sync_copy(xraw_h.at[pl.ds(off, _CH)], val_c)
                pltpu.sync_copy(xind_h.at[pl.ds(off, _CH)], gid_c)

                def inner(kk, carry2):
                    rv = rank_c[pl.ds(kk * 16, 16)]
                    vv = val_c[pl.ds(kk * 16, 16)]
                    gv = gid_c[pl.ds(kk * 16, 16)]
                    ig = (off + kk * 16
                          + lax.broadcasted_iota(jnp.int32, (16,), 0))
                    m = rv < _TOP
                    plsc.store_scatter(tix, [rv], ig, mask=m)
                    plsc.store_scatter(trg, [rv], vv, mask=m)
                    plsc.store_scatter(tgi, [rv], gv, mask=m)
                    return carry2

                lax.fori_loop(0, _CH // 16, inner, 0)
                return carry

            lax.fori_loop(0, _N // _CH, outer, 0)

            stage = core * core_rows + s * _TOP
            pltpu.sync_copy(tix, tixs_h.at[pl.ds(stage, _TOP)])
            pltpu.sync_copy(tgi, tgis_h.at[pl.ds(stage, _TOP)])
            pltpu.sync_copy(trg, rg_h.at[pl.ds(b * _TOP, _TOP)])

        plsc.subcore_barrier()

        for q in range(4):
            c = q * 16 + (15 - s)

            @pl.when(c < nchunks)
            def chunk():
                cbase = core * core_rows + c * _PC
                pltpu.sync_copy(tixs_h.at[pl.ds(cbase, _PC)], idx_c)
                pltpu.sync_copy(tgis_h.at[pl.ds(cbase, _PC)], gidx_c)
                selcp = pltpu.async_copy(fc1_h.at[gidx_c], selb, sem_c)
                nx = _PC // _PCX
                xbufs = (xrow_a, xrow_b)
                xsems = (sem_a, sem_b)
                cps = [None, None]
                cps[0] = pltpu.async_copy(
                    xflat_h.at[idx_c.at[pl.ds(0, _PCX)]], xrow_a, sem_a)
                for i in range(nx):
                    bsl = i % 2
                    if i + 1 < nx:
                        cps[(i + 1) % 2] = pltpu.async_copy(
                            xflat_h.at[idx_c.at[pl.ds((i + 1) * _PCX, _PCX)]],
                            xbufs[(i + 1) % 2], xsems[(i + 1) % 2])
                    cps[bsl].wait()
                    pltpu.sync_copy(xbufs[bsl],
                                    xg_h.at[pl.ds(cbase + i * _PCX, _PCX)])
                selcp.wait()
                pltpu.sync_copy(selb, sel_h.at[pl.ds(cbase, _PC)])

    return k(ranks, xraw, xind, xflat, fc1)


_RB = 800


def _mlp_body(xg_ref, sel_ref, rg_ref, wx_ref, ws_ref, wr_ref, cb_ref,
              t1w_ref, t1b_ref, t2w_ref, t2b_ref, lng_ref, lnb_ref,
              pw_ref, pb_ref, o_ref):
    r = rg_ref[...]
    h1 = jnp.maximum(r * t1w_ref[...] + t1b_ref[...], 0.0)
    remb = (jnp.dot(h1, t2w_ref[...], preferred_element_type=jnp.float32)
            + t2b_ref[...])
    h2 = (jnp.dot(xg_ref[...].astype(jnp.bfloat16),
                  wx_ref[...].astype(jnp.bfloat16),
                  preferred_element_type=jnp.float32)
          + jnp.dot(sel_ref[...], ws_ref[...], preferred_element_type=jnp.float32)
          + jnp.dot(remb, wr_ref[...], preferred_element_type=jnp.float32)
          + cb_ref[...])
    mu = jnp.mean(h2, axis=1, keepdims=True)
    d0 = h2 - mu
    var = jnp.mean(d0 * d0, axis=1, keepdims=True)
    hn = d0 * lax.rsqrt(var + 1e-5) * lng_ref[...] + lnb_ref[...]
    hg = hn * (1.0 / (1.0 + jnp.exp(-1.702 * hn)))
    o_ref[...] = (jnp.dot(hg.astype(jnp.bfloat16),
                          pw_ref[...].astype(jnp.bfloat16),
                          preferred_element_type=jnp.float32)
                  + pb_ref[...])


def _mlp_call(xg, sel, rg2d, wx, ws, wr, cb, t1w, t1b, t2w, t2b,
              lng, lnb, pw, pb):
    full = lambda shape: pl.BlockSpec(shape, lambda i: tuple(0 for _ in shape))
    return pl.pallas_call(
        _mlp_body,
        grid=(_RTOT // _RB,),
        in_specs=[
            pl.BlockSpec((_RB, _D), lambda i: (i, 0)),
            pl.BlockSpec((_RB, _EMB), lambda i: (i, 0)),
            pl.BlockSpec((_RB, 1), lambda i: (i, 0)),
            full((_D, _D)),
            full((_EMB, _D)),
            full((_EMB, _D)),
            full((1, _D)),
            full((1, 50)),
            full((1, 50)),
            full((50, _EMB)),
            full((1, _EMB)),
            full((1, _D)),
            full((1, _D)),
            full((_D, _D)),
            full((1, _D)),
        ],
        out_specs=pl.BlockSpec((_RB, _D), lambda i: (i, 0)),
        out_shape=jax.ShapeDtypeStruct((_RTOT, _D), jnp.float32),
    )(xg, sel, rg2d, wx, ws, wr, cb, t1w, t1b, t2w, t2b, lng, lnb, pw, pb)


def kernel(x, x_raw, x_indices, grn_emb, ppi_emb, fc1_w, fc1_b, t1_w, t1_b,
           t2_w, t2_b, cat_fc_w, cat_fc_b, ln_g, ln_b, proj_w, proj_b):
    ranks = _rank_call(x_raw)
    fc1_out = _fc1_call(grn_emb, ppi_emb, fc1_w, fc1_b.reshape(1, _EMB))
    xg, sel, rg, _, _ = _sc_select_gather(
        ranks.reshape(-1),
        x_raw.reshape(-1),
        x_indices.reshape(-1),
        x.reshape(_B * _N, _D),
        fc1_out,
    )
    y = _mlp_call(
        xg, sel, rg.reshape(_RTOT, 1),
        cat_fc_w[:_D], cat_fc_w[_D:_D + _EMB], cat_fc_w[_D + _EMB:],
        cat_fc_b.reshape(1, _D), t1_w, t1_b.reshape(1, 50), t2_w,
        t2_b.reshape(1, _EMB), ln_g.reshape(1, _D), ln_b.reshape(1, _D),
        proj_w, proj_b.reshape(1, _D),
    )
    return y.reshape(_B, _TOP, _D)


# speedup vs baseline: 1.5527x; 1.1355x over previous
import functools

import jax
import jax.numpy as jnp
from jax import lax
from jax.experimental import pallas as pl
from jax.experimental.pallas import tpu as pltpu
from jax.experimental.pallas import tpu_sc as plsc

_G = 17911
_B, _N, _D = 4, 8192, 768
_EMB = 128
_TOP = 5000
_RTOT = _B * _TOP

def _mono(v):
    b = lax.bitcast_convert_type(v, jnp.int32)
    return jnp.where(b >= 0, b, jnp.full_like(b, 2147483647) - b)


_KB = 1024
_NB = _N // _KB


def _rank_body(xrow_ref, o_ref, mrow, acc_i, acc_j):
    mrow[0, :] = _mono(xrow_ref[0, 0, :])
    acc_i[...] = jnp.zeros((8, _N), jnp.float32)
    acc_j[...] = jnp.zeros((_N, 8), jnp.float32)
    e_row = (lax.broadcasted_iota(jnp.int32, (8, _KB), 0) == 0
             ).astype(jnp.bfloat16)
    e_col = lax.broadcasted_iota(jnp.int32, (_KB, 8), 1) == 0
    e_col_b = e_col.astype(jnp.bfloat16)
    f_const = e_col.astype(jnp.float32) * float(_KB)
    ii = lax.broadcasted_iota(jnp.int32, (_KB, _KB), 1)
    jj = lax.broadcasted_iota(jnp.int32, (_KB, _KB), 0)
    tie_mask = jj < ii

    def a_loop(a, carry):
        mi = mrow[0, pl.ds(a * _KB, _KB)][None, :]
        mjd = mrow[0, pl.ds(a * _KB, _KB)][:, None]
        cd = ((mjd > mi) | ((mjd == mi) & tie_mask)).astype(jnp.bfloat16)
        acc_i[:, pl.ds(a * _KB, _KB)] += jnp.dot(
            e_row, cd, preferred_element_type=jnp.float32)

        def b_loop(b, carry2):
            mj = mrow[0, pl.ds(b * _KB, _KB)][:, None]
            c = (mj > mi).astype(jnp.bfloat16)
            acc_i[:, pl.ds(a * _KB, _KB)] += jnp.dot(
                e_row, c, preferred_element_type=jnp.float32)
            acc_j[pl.ds(b * _KB, _KB), :] += f_const - jnp.dot(
                c, e_col_b, preferred_element_type=jnp.float32)
            return carry2

        lax.fori_loop(a + 1, _NB, b_loop, 0)
        return carry

    lax.fori_loop(0, _NB, a_loop, 0)
    o_ref[0, 0, :] = (jnp.sum(acc_i[...], axis=0)
                      + jnp.sum(acc_j[...], axis=1)).astype(jnp.int32)


def _rank_call(x_raw):
    x3 = x_raw.reshape(_B, 1, _N)
    out = pl.pallas_call(
        _rank_body,
        grid=(_B,),
        in_specs=[pl.BlockSpec((1, 1, _N), lambda b: (b, 0, 0))],
        out_specs=pl.BlockSpec((1, 1, _N), lambda b: (b, 0, 0)),
        out_shape=jax.ShapeDtypeStruct((_B, 1, _N), jnp.int32),
        scratch_shapes=[
            pltpu.VMEM((1, _N), jnp.int32),
            pltpu.VMEM((8, _N), jnp.float32),
            pltpu.VMEM((_N, 8), jnp.float32),
        ],
    )(x3)
    return out.reshape(_B, _N)


_GB = 1024


def _fc1_body(grn_ref, ppi_ref, w_ref, b_ref, o_ref):
    w = w_ref[...]
    o_ref[...] = (
        jnp.dot(grn_ref[...], w[:_EMB, :], preferred_element_type=jnp.float32)
        + jnp.dot(ppi_ref[...], w[_EMB:, :], preferred_element_type=jnp.float32)
        + b_ref[...]
    )


def _fc1_call(grn, ppi, w, b2d):
    ng = (_G + _GB - 1) // _GB
    return pl.pallas_call(
        _fc1_body,
        grid=(ng,),
        in_specs=[
            pl.BlockSpec((_GB, _EMB), lambda i: (i, 0)),
            pl.BlockSpec((_GB, _EMB), lambda i: (i, 0)),
            pl.BlockSpec((2 * _EMB, _EMB), lambda i: (0, 0)),
            pl.BlockSpec((1, _EMB), lambda i: (0, 0)),
        ],
        out_specs=pl.BlockSpec((_GB, _EMB), lambda i: (i, 0)),
        out_shape=jax.ShapeDtypeStruct((_G, _EMB), jnp.float32),
    )(grn, ppi, w, b2d)


_CH = 2048
_PC = 200
_PCX = 40


def _sc_select_gather(ranks, xraw, xind, xflat, fc1):
    mesh = plsc.VectorSubcoreMesh(core_axis_name="c", subcore_axis_name="s")
    core_rows = 2 * _TOP
    nchunks = core_rows // _PC

    @functools.partial(
        pl.kernel,
        mesh=mesh,
        compiler_params=pltpu.CompilerParams(needs_layout_passes=False),
        out_type=[
            jax.ShapeDtypeStruct((_RTOT, _D), jnp.float32),
            jax.ShapeDtypeStruct((_RTOT, _EMB), jnp.float32),
            jax.ShapeDtypeStruct((_RTOT,), jnp.float32),
            jax.ShapeDtypeStruct((_RTOT,), jnp.int32),
            jax.ShapeDtypeStruct((_RTOT,), jnp.int32),
        ],
        scratch_types=[
            pltpu.VMEM((_CH,), jnp.int32),
            pltpu.VMEM((_CH,), jnp.float32),
            pltpu.VMEM((_CH,), jnp.int32),
            pltpu.VMEM((_TOP,), jnp.int32),
            pltpu.VMEM((_TOP,), jnp.float32),
            pltpu.VMEM((_TOP,), jnp.int32),
            pltpu.VMEM((_PC,), jnp.int32),
            pltpu.VMEM((_PC,), jnp.int32),
            pltpu.VMEM((_PCX, _D), jnp.float32),
            pltpu.VMEM((_PCX, _D), jnp.float32),
            pltpu.VMEM((_PC, _EMB), jnp.float32),
            pltpu.SemaphoreType.DMA,
            pltpu.SemaphoreType.DMA,
            pltpu.SemaphoreType.DMA,
        ],
    )
    def k(ranks_h, xraw_h, xind_h, xflat_h, fc1_h,
          xg_h, sel_h, rg_h, tixs_h, tgis_h,
          rank_c, val_c, gid_c, tix, trg, tgi, idx_c, gidx_c,
          xrow_a, xrow_b, selb, sem_a, sem_b, sem_c):
        core = lax.axis_index("c")
        s = lax.axis_index("s")

        @pl.when(s < 2)
        def phase1():
            b = core * 2 + s
            base = b * _N

            def outer(cc, carry):
                off = base + cc * _CH
                pltpu.sync_copy(ranks_h.at[pl.ds(off, _CH)], rank_c)
                pltpu.sync_copy(xraw_h.at[pl.ds(off, _CH)], val_c)
                pltpu.sync_copy(xind_h.at[pl.ds(off, _CH)], gid_c)

                def inner(kk, carry2):
                    rv = rank_c[pl.ds(kk * 16, 16)]
                    vv = val_c[pl.ds(kk * 16, 16)]
                    gv = gid_c[pl.ds(kk * 16, 16)]
                    ig = (off + kk * 16
                          + lax.broadcasted_iota(jnp.int32, (16,), 0))
                    m = rv < _TOP
                    plsc.store_scatter(tix, [rv], ig, mask=m)
                    plsc.store_scatter(trg, [rv], vv, mask=m)
                    plsc.store_scatter(tgi, [rv], gv, mask=m)
                    return carry2

                lax.fori_loop(0, _CH // 16, inner, 0)
                return carry

            lax.fori_loop(0, _N // _CH, outer, 0)

            stage = core * core_rows + s * _TOP
            pltpu.sync_copy(tix, tixs_h.at[pl.ds(stage, _TOP)])
            pltpu.sync_copy(tgi, tgis_h.at[pl.ds(stage, _TOP)])
            pltpu.sync_copy(trg, rg_h.at[pl.ds(b * _TOP, _TOP)])

        plsc.subcore_barrier()

        for q in range(4):
            c = q * 16 + (15 - s)

            @pl.when(c < nchunks)
            def chunk():
                cbase = core * core_rows + c * _PC
                pltpu.sync_copy(tixs_h.at[pl.ds(cbase, _PC)], idx_c)
                pltpu.sync_copy(tgis_h.at[pl.ds(cbase, _PC)], gidx_c)
                selcp = pltpu.async_copy(fc1_h.at[gidx_c], selb, sem_c)
                nx = _PC // _PCX
                xbufs = (xrow_a, xrow_b)
                xsems = (sem_a, sem_b)
                cps = [None, None]
                cps[0] = pltpu.async_copy(
                    xflat_h.at[idx_c.at[pl.ds(0, _PCX)]], xrow_a, sem_a)
                for i in range(nx):
                    bsl = i % 2
                    if i + 1 < nx:
                        cps[(i + 1) % 2] = pltpu.async_copy(
                            xflat_h.at[idx_c.at[pl.ds((i + 1) * _PCX, _PCX)]],
                            xbufs[(i + 1) % 2], xsems[(i + 1) % 2])
                    cps[bsl].wait()
                    pltpu.sync_copy(xbufs[bsl],
                                    xg_h.at[pl.ds(cbase + i * _PCX, _PCX)])
                selcp.wait()
                pltpu.sync_copy(selb, sel_h.at[pl.ds(cbase, _PC)])

    return k(ranks, xraw, xind, xflat, fc1)


_RB = 800


def _mlp_body(xg_ref, sel_ref, rg_ref, wx_ref, ws_ref, wr_ref, cb_ref,
              t1w_ref, t1b_ref, t2w_ref, t2b_ref, lng_ref, lnb_ref,
              pw_ref, pb_ref, o_ref):
    r = rg_ref[...]
    h1 = jnp.maximum(r * t1w_ref[...] + t1b_ref[...], 0.0)
    remb = (jnp.dot(h1, t2w_ref[...], preferred_element_type=jnp.float32)
            + t2b_ref[...])
    h2 = (jnp.dot(xg_ref[...].astype(jnp.bfloat16),
                  wx_ref[...].astype(jnp.bfloat16),
                  preferred_element_type=jnp.float32)
          + jnp.dot(sel_ref[...], ws_ref[...], preferred_element_type=jnp.float32)
          + jnp.dot(remb, wr_ref[...], preferred_element_type=jnp.float32)
          + cb_ref[...])
    mu = jnp.mean(h2, axis=1, keepdims=True)
    d0 = h2 - mu
    var = jnp.mean(d0 * d0, axis=1, keepdims=True)
    hn = d0 * lax.rsqrt(var + 1e-5) * lng_ref[...] + lnb_ref[...]
    hg = hn * (1.0 / (1.0 + jnp.exp(-1.702 * hn)))
    o_ref[...] = (jnp.dot(hg.astype(jnp.bfloat16),
                          pw_ref[...].astype(jnp.bfloat16),
                          preferred_element_type=jnp.float32)
                  + pb_ref[...])


def _mlp_call(xg, sel, rg2d, wx, ws, wr, cb, t1w, t1b, t2w, t2b,
              lng, lnb, pw, pb):
    full = lambda shape: pl.BlockSpec(shape, lambda i: tuple(0 for _ in shape))
    return pl.pallas_call(
        _mlp_body,
        grid=(_RTOT // _RB,),
        in_specs=[
            pl.BlockSpec((_RB, _D), lambda i: (i, 0)),
            pl.BlockSpec((_RB, _EMB), lambda i: (i, 0)),
            pl.BlockSpec((_RB, 1), lambda i: (i, 0)),
            full((_D, _D)),
            full((_EMB, _D)),
            full((_EMB, _D)),
            full((1, _D)),
            full((1, 50)),
            full((1, 50)),
            full((50, _EMB)),
            full((1, _EMB)),
            full((1, _D)),
            full((1, _D)),
            full((_D, _D)),
            full((1, _D)),
        ],
        out_specs=pl.BlockSpec((_RB, _D), lambda i: (i, 0)),
        out_shape=jax.ShapeDtypeStruct((_RTOT, _D), jnp.float32),
    )(xg, sel, rg2d, wx, ws, wr, cb, t1w, t1b, t2w, t2b, lng, lnb, pw, pb)


def kernel(x, x_raw, x_indices, grn_emb, ppi_emb, fc1_w, fc1_b, t1_w, t1_b,
           t2_w, t2_b, cat_fc_w, cat_fc_b, ln_g, ln_b, proj_w, proj_b):
    ranks = _rank_call(x_raw)
    fc1_out = _fc1_call(grn_emb, ppi_emb, fc1_w, fc1_b.reshape(1, _EMB))
    xg, sel, rg, _, _ = _sc_select_gather(
        ranks.reshape(-1),
        x_raw.reshape(-1),
        x_indices.reshape(-1),
        x.reshape(_B * _N, _D),
        fc1_out,
    )
    y = _mlp_call(
        xg, sel, rg.reshape(_RTOT, 1),
        cat_fc_w[:_D], cat_fc_w[_D:_D + _EMB], cat_fc_w[_D + _EMB:],
        cat_fc_b.reshape(1, _D), t1_w, t1_b.reshape(1, 50), t2_w,
        t2_b.reshape(1, _EMB), ln_g.reshape(1, _D), ln_b.reshape(1, _D),
        proj_w, proj_b.reshape(1, _D),
    )
    return y.reshape(_B, _TOP, _D)


# speedup vs baseline: 1.7259x; 1.1116x over previous
import functools

import jax
import jax.numpy as jnp
from jax import lax
from jax.experimental import pallas as pl
from jax.experimental.pallas import tpu as pltpu
from jax.experimental.pallas import tpu_sc as plsc

_G = 17911
_B, _N, _D = 4, 8192, 768
_EMB = 128
_TOP = 5000
_RTOT = _B * _TOP

def _mono(v):
    b = lax.bitcast_convert_type(v, jnp.int32)
    return jnp.where(b >= 0, b, jnp.full_like(b, 2147483647) - b)


_KB = 1024
_NB = _N // _KB
_GBF = 4480


def _rank_body(xrow_ref, grn_ref, ppi_ref, w_ref, b_ref, o_ref, f_ref,
               mrow, macc):
    w = w_ref[...]
    f_ref[...] = (
        jnp.dot(grn_ref[...], w[:_EMB, :], preferred_element_type=jnp.float32)
        + jnp.dot(ppi_ref[...], w[_EMB:, :], preferred_element_type=jnp.float32)
        + b_ref[...]
    )

    mrow[0, :] = _mono(xrow_ref[0, 0, :])
    ii = lax.broadcasted_iota(jnp.int32, (_KB, _KB), 1)
    jj = lax.broadcasted_iota(jnp.int32, (_KB, _KB), 0)
    tie_mask = jj < ii

    def a_loop(a, carry):
        mi = mrow[0, pl.ds(a * _KB, _KB)][None, :]
        macc[...] = ((mrow[0, pl.ds(a * _KB, _KB)][:, None] > mi)
                     | ((mrow[0, pl.ds(a * _KB, _KB)][:, None] == mi)
                        & tie_mask)).astype(jnp.float32)

        def lo_loop(b, carry2):
            mj = mrow[0, pl.ds(b * _KB, _KB)][:, None]
            macc[...] += (mj >= mi).astype(jnp.float32)
            return carry2

        def hi_loop(b, carry2):
            mj = mrow[0, pl.ds(b * _KB, _KB)][:, None]
            macc[...] += (mj > mi).astype(jnp.float32)
            return carry2

        lax.fori_loop(0, a, lo_loop, 0)
        lax.fori_loop(a + 1, _NB, hi_loop, 0)
        o_ref[0, 0, pl.ds(a * _KB, _KB)] = jnp.sum(
            macc[...], axis=0).astype(jnp.int32)
        return carry

    lax.fori_loop(0, _NB, a_loop, 0)


def _rank_fc1_call(x_raw, grn, ppi, w, b2d):
    x3 = x_raw.reshape(_B, 1, _N)
    ranks, fc1 = pl.pallas_call(
        _rank_body,
        grid=(_B,),
        in_specs=[
            pl.BlockSpec((1, 1, _N), lambda b: (b, 0, 0)),
            pl.BlockSpec((_GBF, _EMB), lambda b: (b, 0)),
            pl.BlockSpec((_GBF, _EMB), lambda b: (b, 0)),
            pl.BlockSpec((2 * _EMB, _EMB), lambda b: (0, 0)),
            pl.BlockSpec((1, _EMB), lambda b: (0, 0)),
        ],
        out_specs=[
            pl.BlockSpec((1, 1, _N), lambda b: (b, 0, 0)),
            pl.BlockSpec((_GBF, _EMB), lambda b: (b, 0)),
        ],
        out_shape=[
            jax.ShapeDtypeStruct((_B, 1, _N), jnp.int32),
            jax.ShapeDtypeStruct((_G, _EMB), jnp.float32),
        ],
        scratch_shapes=[
            pltpu.VMEM((1, _N), jnp.int32),
            pltpu.VMEM((_KB, _KB), jnp.float32),
        ],
    )(x3, grn, ppi, w, b2d)
    return ranks.reshape(_B, _N), fc1


_CH = 2048
_PC = 200
_PCX = 40


def _sc_select_gather(ranks, xraw, xind, xflat, fc1):
    mesh = plsc.VectorSubcoreMesh(core_axis_name="c", subcore_axis_name="s")
    core_rows = 2 * _TOP
    nchunks = core_rows // _PC

    @functools.partial(
        pl.kernel,
        mesh=mesh,
        compiler_params=pltpu.CompilerParams(needs_layout_passes=False),
        out_type=[
            jax.ShapeDtypeStruct((_RTOT, _D), jnp.float32),
            jax.ShapeDtypeStruct((_RTOT, _EMB), jnp.float32),
            jax.ShapeDtypeStruct((_RTOT,), jnp.float32),
            jax.ShapeDtypeStruct((_RTOT,), jnp.int32),
            jax.ShapeDtypeStruct((_RTOT,), jnp.int32),
        ],
        scratch_types=[
            pltpu.VMEM((_CH,), jnp.int32),
            pltpu.VMEM((_CH,), jnp.float32),
            pltpu.VMEM((_CH,), jnp.int32),
            pltpu.VMEM((_TOP,), jnp.int32),
            pltpu.VMEM((_TOP,), jnp.float32),
            pltpu.VMEM((_TOP,), jnp.int32),
            pltpu.VMEM((_PC,), jnp.int32),
            pltpu.VMEM((_PC,), jnp.int32),
            pltpu.VMEM((_PCX, _D), jnp.float32),
            pltpu.VMEM((_PCX, _D), jnp.float32),
            pltpu.VMEM((_PC, _EMB), jnp.float32),
            pltpu.SemaphoreType.DMA,
            pltpu.SemaphoreType.DMA,
            pltpu.SemaphoreType.DMA,
        ],
    )
    def k(ranks_h, xraw_h, xind_h, xflat_h, fc1_h,
          xg_h, sel_h, rg_h, tixs_h, tgis_h,
          rank_c, val_c, gid_c, tix, trg, tgi, idx_c, gidx_c,
          xrow_a, xrow_b, selb, sem_a, sem_b, sem_c):
        core = lax.axis_index("c")
        s = lax.axis_index("s")

        @pl.when(s < 2)
        def phase1():
            b = core * 2 + s
            base = b * _N

            def outer(cc, carry):
                off = base + cc * _CH
                pltpu.sync_copy(ranks_h.at[pl.ds(off, _CH)], rank_c)
                pltpu.sync_copy(xraw_h.at[pl.ds(off, _CH)], val_c)
                pltpu.sync_copy(xind_h.at[pl.ds(off, _CH)], gid_c)

                def inner(kk, carry2):
                    rv = rank_c[pl.ds(kk * 16, 16)]
                    vv = val_c[pl.ds(kk * 16, 16)]
                    gv = gid_c[pl.ds(kk * 16, 16)]
                    ig = (off + kk * 16
                          + lax.broadcasted_iota(jnp.int32, (16,), 0))
                    m = rv < _TOP
                    plsc.store_scatter(tix, [rv], ig, mask=m)
                    plsc.store_scatter(trg, [rv], vv, mask=m)
                    plsc.store_scatter(tgi, [rv], gv, mask=m)
                    return carry2

                lax.fori_loop(0, _CH // 16, inner, 0)
                return carry

            lax.fori_loop(0, _N // _CH, outer, 0)

            stage = core * core_rows + s * _TOP
            pltpu.sync_copy(tix, tixs_h.at[pl.ds(stage, _TOP)])
            pltpu.sync_copy(tgi, tgis_h.at[pl.ds(stage, _TOP)])
            pltpu.sync_copy(trg, rg_h.at[pl.ds(b * _TOP, _TOP)])

        plsc.subcore_barrier()

        for q in range(4):
            c = q * 16 + (15 - s)

            @pl.when(c < nchunks)
            def chunk():
                cbase = core * core_rows + c * _PC
                pltpu.sync_copy(tixs_h.at[pl.ds(cbase, _PC)], idx_c)
                pltpu.sync_copy(tgis_h.at[pl.ds(cbase, _PC)], gidx_c)
                selcp = pltpu.async_copy(fc1_h.at[gidx_c], selb, sem_c)
                nx = _PC // _PCX
                xbufs = (xrow_a, xrow_b)
                xsems = (sem_a, sem_b)
                cps = [None, None]
                cps[0] = pltpu.async_copy(
                    xflat_h.at[idx_c.at[pl.ds(0, _PCX)]], xrow_a, sem_a)
                for i in range(nx):
                    bsl = i % 2
                    if i + 1 < nx:
                        cps[(i + 1) % 2] = pltpu.async_copy(
                            xflat_h.at[idx_c.at[pl.ds((i + 1) * _PCX, _PCX)]],
                            xbufs[(i + 1) % 2], xsems[(i + 1) % 2])
                    cps[bsl].wait()
                    pltpu.sync_copy(xbufs[bsl],
                                    xg_h.at[pl.ds(cbase + i * _PCX, _PCX)])
                selcp.wait()
                pltpu.sync_copy(selb, sel_h.at[pl.ds(cbase, _PC)])

    return k(ranks, xraw, xind, xflat, fc1)


_RB = 800


def _mlp_body(xg_ref, sel_ref, rg_ref, wx_ref, ws_ref, wr_ref, cb_ref,
              t1w_ref, t1b_ref, t2w_ref, t2b_ref, lng_ref, lnb_ref,
              pw_ref, pb_ref, o_ref):
    r = rg_ref[...]
    h1 = jnp.maximum(r * t1w_ref[...] + t1b_ref[...], 0.0)
    remb = (jnp.dot(h1, t2w_ref[...], preferred_element_type=jnp.float32)
            + t2b_ref[...])
    h2 = (jnp.dot(xg_ref[...].astype(jnp.bfloat16),
                  wx_ref[...].astype(jnp.bfloat16),
                  preferred_element_type=jnp.float32)
          + jnp.dot(sel_ref[...], ws_ref[...], preferred_element_type=jnp.float32)
          + jnp.dot(remb, wr_ref[...], preferred_element_type=jnp.float32)
          + cb_ref[...])
    mu = jnp.mean(h2, axis=1, keepdims=True)
    d0 = h2 - mu
    var = jnp.mean(d0 * d0, axis=1, keepdims=True)
    hn = d0 * lax.rsqrt(var + 1e-5) * lng_ref[...] + lnb_ref[...]
    hg = hn * (1.0 / (1.0 + jnp.exp(-1.702 * hn)))
    o_ref[...] = (jnp.dot(hg.astype(jnp.bfloat16),
                          pw_ref[...].astype(jnp.bfloat16),
                          preferred_element_type=jnp.float32)
                  + pb_ref[...])


def _mlp_call(xg, sel, rg2d, wx, ws, wr, cb, t1w, t1b, t2w, t2b,
              lng, lnb, pw, pb):
    full = lambda shape: pl.BlockSpec(shape, lambda i: tuple(0 for _ in shape))
    return pl.pallas_call(
        _mlp_body,
        grid=(_RTOT // _RB,),
        in_specs=[
            pl.BlockSpec((_RB, _D), lambda i: (i, 0)),
            pl.BlockSpec((_RB, _EMB), lambda i: (i, 0)),
            pl.BlockSpec((_RB, 1), lambda i: (i, 0)),
            full((_D, _D)),
            full((_EMB, _D)),
            full((_EMB, _D)),
            full((1, _D)),
            full((1, 50)),
            full((1, 50)),
            full((50, _EMB)),
            full((1, _EMB)),
            full((1, _D)),
            full((1, _D)),
            full((_D, _D)),
            full((1, _D)),
        ],
        out_specs=pl.BlockSpec((_RB, _D), lambda i: (i, 0)),
        out_shape=jax.ShapeDtypeStruct((_RTOT, _D), jnp.float32),
    )(xg, sel, rg2d, wx, ws, wr, cb, t1w, t1b, t2w, t2b, lng, lnb, pw, pb)


def kernel(x, x_raw, x_indices, grn_emb, ppi_emb, fc1_w, fc1_b, t1_w, t1_b,
           t2_w, t2_b, cat_fc_w, cat_fc_b, ln_g, ln_b, proj_w, proj_b):
    ranks, fc1_out = _rank_fc1_call(x_raw, grn_emb, ppi_emb, fc1_w,
                                    fc1_b.reshape(1, _EMB))
    xg, sel, rg, _, _ = _sc_select_gather(
        ranks.reshape(-1),
        x_raw.reshape(-1),
        x_indices.reshape(-1),
        x.reshape(_B * _N, _D),
        fc1_out,
    )
    y = _mlp_call(
        xg, sel, rg.reshape(_RTOT, 1),
        cat_fc_w[:_D], cat_fc_w[_D:_D + _EMB], cat_fc_w[_D + _EMB:],
        cat_fc_b.reshape(1, _D), t1_w, t1_b.reshape(1, 50), t2_w,
        t2_b.reshape(1, _EMB), ln_g.reshape(1, _D), ln_b.reshape(1, _D),
        proj_w, proj_b.reshape(1, _D),
    )
    return y.reshape(_B, _TOP, _D)


# speedup vs baseline: 1.7310x; 1.0029x over previous
import functools

import jax
import jax.numpy as jnp
from jax import lax
from jax.experimental import pallas as pl
from jax.experimental.pallas import tpu as pltpu
from jax.experimental.pallas import tpu_sc as plsc

_G = 17911
_B, _N, _D = 4, 8192, 768
_EMB = 128
_TOP = 5000
_RTOT = _B * _TOP

def _mono(v):
    b = lax.bitcast_convert_type(v, jnp.int32)
    return jnp.where(b >= 0, b, jnp.full_like(b, 2147483647) - b)


_KB = 1024
_NB = _N // _KB
_GBF = 4480


def _rank_body(xrow_ref, grn_ref, ppi_ref, w_ref, b_ref, o_ref, f_ref,
               mrow, macc):
    w = w_ref[...]
    f_ref[...] = (
        jnp.dot(grn_ref[...], w[:_EMB, :], preferred_element_type=jnp.float32)
        + jnp.dot(ppi_ref[...], w[_EMB:, :], preferred_element_type=jnp.float32)
        + b_ref[...]
    )

    mrow[0, :] = _mono(xrow_ref[0, 0, :])
    ii = lax.broadcasted_iota(jnp.int32, (_KB, _KB), 1)
    jj = lax.broadcasted_iota(jnp.int32, (_KB, _KB), 0)
    tie_mask = jj < ii

    def a_loop(a, carry):
        mi = mrow[0, pl.ds(a * _KB, _KB)][None, :]
        macc[...] = ((mrow[0, pl.ds(a * _KB, _KB)][:, None] > mi)
                     | ((mrow[0, pl.ds(a * _KB, _KB)][:, None] == mi)
                        & tie_mask)).astype(jnp.float32)

        def lo_loop(b, carry2):
            mj = mrow[0, pl.ds(b * _KB, _KB)][:, None]
            macc[...] += (mj >= mi).astype(jnp.float32)
            return carry2

        def hi_loop(b, carry2):
            mj = mrow[0, pl.ds(b * _KB, _KB)][:, None]
            macc[...] += (mj > mi).astype(jnp.float32)
            return carry2

        lax.fori_loop(0, a, lo_loop, 0)
        lax.fori_loop(a + 1, _NB, hi_loop, 0)
        o_ref[0, 0, pl.ds(a * _KB, _KB)] = jnp.sum(
            macc[...], axis=0).astype(jnp.int32)
        return carry

    lax.fori_loop(0, _NB, a_loop, 0)


def _rank_fc1_call(x_raw, grn, ppi, w, b2d):
    x3 = x_raw.reshape(_B, 1, _N)
    ranks, fc1 = pl.pallas_call(
        _rank_body,
        grid=(_B,),
        in_specs=[
            pl.BlockSpec((1, 1, _N), lambda b: (b, 0, 0)),
            pl.BlockSpec((_GBF, _EMB), lambda b: (b, 0)),
            pl.BlockSpec((_GBF, _EMB), lambda b: (b, 0)),
            pl.BlockSpec((2 * _EMB, _EMB), lambda b: (0, 0)),
            pl.BlockSpec((1, _EMB), lambda b: (0, 0)),
        ],
        out_specs=[
            pl.BlockSpec((1, 1, _N), lambda b: (b, 0, 0)),
            pl.BlockSpec((_GBF, _EMB), lambda b: (b, 0)),
        ],
        out_shape=[
            jax.ShapeDtypeStruct((_B, 1, _N), jnp.int32),
            jax.ShapeDtypeStruct((_G, _EMB), jnp.float32),
        ],
        scratch_shapes=[
            pltpu.VMEM((1, _N), jnp.int32),
            pltpu.VMEM((_KB, _KB), jnp.float32),
        ],
    )(x3, grn, ppi, w, b2d)
    return ranks.reshape(_B, _N), fc1


_CH = 2048
_PC = 200
_PCX = 40


def _sc_select_gather(ranks, xraw, xind, xflat, fc1):
    mesh = plsc.VectorSubcoreMesh(core_axis_name="c", subcore_axis_name="s")
    core_rows = 2 * _TOP
    nchunks = core_rows // _PC

    @functools.partial(
        pl.kernel,
        mesh=mesh,
        compiler_params=pltpu.CompilerParams(needs_layout_passes=False),
        out_type=[
            jax.ShapeDtypeStruct((_RTOT, _D), jnp.float32),
            jax.ShapeDtypeStruct((_RTOT, _EMB), jnp.float32),
            jax.ShapeDtypeStruct((_RTOT,), jnp.float32),
            jax.ShapeDtypeStruct((_RTOT,), jnp.int32),
            jax.ShapeDtypeStruct((_RTOT,), jnp.int32),
        ],
        scratch_types=[
            pltpu.VMEM((_CH,), jnp.int32),
            pltpu.VMEM((_CH,), jnp.float32),
            pltpu.VMEM((_CH,), jnp.int32),
            pltpu.VMEM((_TOP,), jnp.int32),
            pltpu.VMEM((_TOP,), jnp.float32),
            pltpu.VMEM((_TOP,), jnp.int32),
            pltpu.VMEM((_PC,), jnp.int32),
            pltpu.VMEM((_PC,), jnp.int32),
            pltpu.VMEM((_PCX, _D), jnp.float32),
            pltpu.VMEM((_PCX, _D), jnp.float32),
            pltpu.VMEM((_PC, _EMB), jnp.float32),
            pltpu.SemaphoreType.DMA,
            pltpu.SemaphoreType.DMA,
            pltpu.SemaphoreType.DMA,
        ],
    )
    def k(ranks_h, xraw_h, xind_h, xflat_h, fc1_h,
          xg_h, sel_h, rg_h, tixs_h, tgis_h,
          rank_c, val_c, gid_c, tix, trg, tgi, idx_c, gidx_c,
          xrow_a, xrow_b, selb, sem_a, sem_b, sem_c):
        core = lax.axis_index("c")
        s = lax.axis_index("s")

        @pl.when(s < 2)
        def phase1():
            b = core * 2 + s
            base = b * _N

            def outer(cc, carry):
                off = base + cc * _CH
                pltpu.sync_copy(ranks_h.at[pl.ds(off, _CH)], rank_c)
                pltpu.sync_copy(xraw_h.at[pl.ds(off, _CH)], val_c)
                pltpu.sync_copy(xind_h.at[pl.ds(off, _CH)], gid_c)

                def inner(kk, carry2):
                    rv = rank_c[pl.ds(kk * 16, 16)]
                    vv = val_c[pl.ds(kk * 16, 16)]
                    gv = gid_c[pl.ds(kk * 16, 16)]
                    ig = (off + kk * 16
                          + lax.broadcasted_iota(jnp.int32, (16,), 0))
                    m = rv < _TOP
                    plsc.store_scatter(tix, [rv], ig, mask=m)
                    plsc.store_scatter(trg, [rv], vv, mask=m)
                    plsc.store_scatter(tgi, [rv], gv, mask=m)
                    return carry2

                lax.fori_loop(0, _CH // 16, inner, 0)
                return carry

            lax.fori_loop(0, _N // _CH, outer, 0)

            stage = core * core_rows + s * _TOP
            pltpu.sync_copy(tix, tixs_h.at[pl.ds(stage, _TOP)])
            pltpu.sync_copy(tgi, tgis_h.at[pl.ds(stage, _TOP)])
            pltpu.sync_copy(trg, rg_h.at[pl.ds(b * _TOP, _TOP)])

        plsc.subcore_barrier()

        for q in range(4):
            c = q * 16 + (15 - s)

            @pl.when(c < nchunks)
            def chunk():
                cbase = core * core_rows + c * _PC
                pltpu.sync_copy(tixs_h.at[pl.ds(cbase, _PC)], idx_c)
                pltpu.sync_copy(tgis_h.at[pl.ds(cbase, _PC)], gidx_c)
                selcp = pltpu.async_copy(fc1_h.at[gidx_c], selb, sem_c)
                nx = _PC // _PCX
                xbufs = (xrow_a, xrow_b)
                xsems = (sem_a, sem_b)
                cps = [None, None]
                cps[0] = pltpu.async_copy(
                    xflat_h.at[idx_c.at[pl.ds(0, _PCX)]], xrow_a, sem_a)
                for i in range(nx):
                    bsl = i % 2
                    if i + 1 < nx:
                        cps[(i + 1) % 2] = pltpu.async_copy(
                            xflat_h.at[idx_c.at[pl.ds((i + 1) * _PCX, _PCX)]],
                            xbufs[(i + 1) % 2], xsems[(i + 1) % 2])
                    cps[bsl].wait()
                    pltpu.sync_copy(xbufs[bsl],
                                    xg_h.at[pl.ds(cbase + i * _PCX, _PCX)])
                selcp.wait()
                pltpu.sync_copy(selb, sel_h.at[pl.ds(cbase, _PC)])

    return k(ranks, xraw, xind, xflat, fc1)


_RB = 800


def _mlp_body(xg_ref, sel_ref, rg_ref, wx_ref, ws_ref, wr_ref, cb_ref,
              t1w_ref, t1b_ref, t2w_ref, t2b_ref, lng_ref, lnb_ref,
              pw_ref, pb_ref, o_ref):
    r = rg_ref[...]
    h1 = jnp.maximum(r * t1w_ref[...] + t1b_ref[...], 0.0)
    remb = (jnp.dot(h1.astype(jnp.bfloat16),
                    t2w_ref[...].astype(jnp.bfloat16),
                    preferred_element_type=jnp.float32)
            + t2b_ref[...])
    h2 = (jnp.dot(xg_ref[...].astype(jnp.bfloat16),
                  wx_ref[...].astype(jnp.bfloat16),
                  preferred_element_type=jnp.float32)
          + jnp.dot(sel_ref[...].astype(jnp.bfloat16),
                    ws_ref[...].astype(jnp.bfloat16),
                    preferred_element_type=jnp.float32)
          + jnp.dot(remb.astype(jnp.bfloat16),
                    wr_ref[...].astype(jnp.bfloat16),
                    preferred_element_type=jnp.float32)
          + cb_ref[...])
    mu = jnp.mean(h2, axis=1, keepdims=True)
    d0 = h2 - mu
    var = jnp.mean(d0 * d0, axis=1, keepdims=True)
    hn = d0 * lax.rsqrt(var + 1e-5) * lng_ref[...] + lnb_ref[...]
    hg = hn * (1.0 / (1.0 + jnp.exp(-1.702 * hn)))
    o_ref[...] = (jnp.dot(hg.astype(jnp.bfloat16),
                          pw_ref[...].astype(jnp.bfloat16),
                          preferred_element_type=jnp.float32)
                  + pb_ref[...])


def _mlp_call(xg, sel, rg2d, wx, ws, wr, cb, t1w, t1b, t2w, t2b,
              lng, lnb, pw, pb):
    full = lambda shape: pl.BlockSpec(shape, lambda i: tuple(0 for _ in shape))
    return pl.pallas_call(
        _mlp_body,
        grid=(_RTOT // _RB,),
        in_specs=[
            pl.BlockSpec((_RB, _D), lambda i: (i, 0)),
            pl.BlockSpec((_RB, _EMB), lambda i: (i, 0)),
            pl.BlockSpec((_RB, 1), lambda i: (i, 0)),
            full((_D, _D)),
            full((_EMB, _D)),
            full((_EMB, _D)),
            full((1, _D)),
            full((1, 50)),
            full((1, 50)),
            full((50, _EMB)),
            full((1, _EMB)),
            full((1, _D)),
            full((1, _D)),
            full((_D, _D)),
            full((1, _D)),
        ],
        out_specs=pl.BlockSpec((_RB, _D), lambda i: (i, 0)),
        out_shape=jax.ShapeDtypeStruct((_RTOT, _D), jnp.float32),
    )(xg, sel, rg2d, wx, ws, wr, cb, t1w, t1b, t2w, t2b, lng, lnb, pw, pb)


def kernel(x, x_raw, x_indices, grn_emb, ppi_emb, fc1_w, fc1_b, t1_w, t1_b,
           t2_w, t2_b, cat_fc_w, cat_fc_b, ln_g, ln_b, proj_w, proj_b):
    ranks, fc1_out = _rank_fc1_call(x_raw, grn_emb, ppi_emb, fc1_w,
                                    fc1_b.reshape(1, _EMB))
    xg, sel, rg, _, _ = _sc_select_gather(
        ranks.reshape(-1),
        x_raw.reshape(-1),
        x_indices.reshape(-1),
        x.reshape(_B * _N, _D),
        fc1_out,
    )
    y = _mlp_call(
        xg, sel, rg.reshape(_RTOT, 1),
        cat_fc_w[:_D], cat_fc_w[_D:_D + _EMB], cat_fc_w[_D + _EMB:],
        cat_fc_b.reshape(1, _D), t1_w, t1_b.reshape(1, 50), t2_w,
        t2_b.reshape(1, _EMB), ln_g.reshape(1, _D), ln_b.reshape(1, _D),
        proj_w, proj_b.reshape(1, _D),
    )
    return y.reshape(_B, _TOP, _D)
